# Initial kernel scaffold; baseline (speedup 1.0000x reference)
import functools

import jax
import jax.numpy as jnp
from jax import lax
from jax.experimental import pallas as pl
from jax.experimental.pallas import tpu as pltpu
from jax.experimental.pallas import tpu_sc as plsc

_N = 10000
_D = 128
_E = 320000
_NC = 2
_NS = 16
_NW = _NC * _NS
_EPW = _E // _NW
_CHUNK = 80
_NCHUNK = _EPW // _CHUNK
_RPS = _N // _NS
_WB = 125
_LANES = 16


def _seg_body(h_hbm, src_hbm, dst_hbm, out_hbm, sidx, didx, rows, zbuf, acc, sem):
    cid = lax.axis_index("c")
    sid = lax.axis_index("s")
    wid = sid * _NC + cid

    def zrow(i, carry):
        for j in range(_D // _LANES):
            zbuf[i, pl.ds(j * _LANES, _LANES)] = jnp.zeros((_LANES,), jnp.float32)
        return carry

    lax.fori_loop(0, _WB, zrow, 0)
    for j in range(_RPS // _WB):
        pltpu.sync_copy(zbuf, acc.at[pl.ds(sid * _RPS + j * _WB, _WB)])
    plsc.subcore_barrier()

    ebase = wid * _EPW

    def body(c, carry):
        base = pl.multiple_of(ebase + c * _CHUNK, 8)
        pltpu.sync_copy(src_hbm.at[pl.ds(base, _CHUNK)], sidx)
        pltpu.sync_copy(dst_hbm.at[pl.ds(base, _CHUNK)], didx)
        pltpu.async_copy(h_hbm.at[sidx], rows, sem).wait()
        pltpu.sync_copy(rows, acc.at[didx], add=True)
        return carry

    lax.fori_loop(0, _NCHUNK, body, 0)
    plsc.subcore_barrier()

    obase = cid * _N + sid * _RPS
    for j in range(_RPS // _WB):
        pltpu.sync_copy(acc.at[pl.ds(sid * _RPS + j * _WB, _WB)], zbuf)
        pltpu.sync_copy(zbuf, out_hbm.at[pl.ds(obase + j * _WB, _WB)])


_segsum = functools.partial(
    pl.kernel,
    out_type=jax.ShapeDtypeStruct((_NC * _N, _D), jnp.float32),
    mesh=plsc.VectorSubcoreMesh(core_axis_name="c", subcore_axis_name="s"),
    scratch_types=[
        pltpu.VMEM((_CHUNK,), jnp.int32),
        pltpu.VMEM((_CHUNK,), jnp.int32),
        pltpu.VMEM((_CHUNK, _D), jnp.float32),
        pltpu.VMEM((_WB, _D), jnp.float32),
        pltpu.VMEM_SHARED((_N, _D), jnp.float32),
        pltpu.SemaphoreType.DMA,
    ],
)(_seg_body)


def _mlp1_body(s_ref, x_ref, p_ref, W1_ref, b1_ref, g1_ref, be1_ref, W2_ref,
               b2_ref, o_ref):
    h = x_ref[...] * s_ref[0, 0] + p_ref[0] + p_ref[1]
    z = jnp.dot(h, W1_ref[...], preferred_element_type=jnp.float32) + b1_ref[...]
    mu = jnp.mean(z, axis=0, keepdims=True)
    zc = z - mu
    var = jnp.mean(zc * zc, axis=0, keepdims=True)
    z = zc * lax.rsqrt(var + 1e-5) * g1_ref[...] + be1_ref[...]
    z = jnp.maximum(z, 0.0)
    z = jnp.dot(z, W2_ref[...], preferred_element_type=jnp.float32) + b2_ref[...]
    o_ref[...] = jnp.maximum(z, 0.0)


def _mlp2_body(s_ref, h_ref, p_ref, W3_ref, b3_ref, g2_ref, be2_ref, W4_ref,
               b4_ref, g3_ref, be3_ref, Wfc_ref, bfc_ref, o_ref):
    h = h_ref[...] * s_ref[0, 0] + p_ref[0] + p_ref[1]
    z = jnp.dot(h, W3_ref[...], preferred_element_type=jnp.float32) + b3_ref[...]
    mu = jnp.mean(z, axis=0, keepdims=True)
    zc = z - mu
    var = jnp.mean(zc * zc, axis=0, keepdims=True)
    z = zc * lax.rsqrt(var + 1e-5) * g2_ref[...] + be2_ref[...]
    z = jnp.maximum(z, 0.0)
    z = jnp.dot(z, W4_ref[...], preferred_element_type=jnp.float32) + b4_ref[...]
    mu2 = jnp.mean(z, axis=0, keepdims=True)
    zc2 = z - mu2
    var2 = jnp.mean(zc2 * zc2, axis=0, keepdims=True)
    z = zc2 * lax.rsqrt(var2 + 1e-5) * g3_ref[...] + be3_ref[...]
    z = jnp.maximum(z, 0.0)
    o_ref[...] = (jnp.dot(z, Wfc_ref[...], preferred_element_type=jnp.float32)
                  + bfc_ref[...])


def kernel(x, edge_index, eps1, W1, b1, g1, be1, W2, b2, eps2, W3, b3, g2,
           be2, W4, b4, g3, be3, Wfc, bfc):
    src = edge_index[0]
    dst = edge_index[1]

    p1 = _segsum(x, src, dst).reshape(2, _N, _D)
    h1 = pl.pallas_call(
        _mlp1_body,
        out_shape=jax.ShapeDtypeStruct((_N, _D), jnp.float32),
    )(
        (1.0 + eps1).reshape(1, 1), x, p1, W1, b1.reshape(1, _D),
        g1.reshape(1, _D), be1.reshape(1, _D), W2, b2.reshape(1, _D),
    )

    p2 = _segsum(h1, src, dst).reshape(2, _N, _D)
    out = pl.pallas_call(
        _mlp2_body,
        out_shape=jax.ShapeDtypeStruct((_N, Wfc.shape[1]), jnp.float32),
    )(
        (1.0 + eps2).reshape(1, 1), h1, p2, W3, b3.reshape(1, _D),
        g2.reshape(1, _D), be2.reshape(1, _D), W4, b4.reshape(1, _D),
        g3.reshape(1, _D), be3.reshape(1, _D), Wfc,
        bfc.reshape(1, bfc.shape[0]),
    )
    return out


# speedup vs baseline: 4.6895x; 4.6895x over previous
import functools

import jax
import jax.numpy as jnp
from jax import lax
from jax.experimental import pallas as pl
from jax.experimental.pallas import tpu as pltpu
from jax.experimental.pallas import tpu_sc as plsc

_N = 10000
_D = 128
_E = 320000
_NC = 2
_NS = 16
_NW = _NC * _NS
_EPW = _E // _NW
_CHUNK = 80
_NCHUNK = _EPW // _CHUNK
_OWN = 624
_WB = 104
_REM = _N - _NS * _OWN
_LANES = 16


def _seg_body(h_hbm, src_hbm, dst_hbm, out_hbm, sidx, didx, rows, zbuf, acc, sem):
    cid = lax.axis_index("c")
    sid = lax.axis_index("s")
    wid = sid * _NC + cid

    def zrow(i, carry):
        for j in range(_D // _LANES):
            zbuf[i, pl.ds(j * _LANES, _LANES)] = jnp.zeros((_LANES,), jnp.float32)
        return carry

    lax.fori_loop(0, _WB, zrow, 0)
    rbase = sid * _OWN
    for j in range(_OWN // _WB):
        off = pl.multiple_of(rbase + j * _WB, 8)
        pltpu.sync_copy(zbuf, acc.at[pl.ds(off, _WB)])

    @pl.when(sid == _NS - 1)
    def _zero_rem():
        pltpu.sync_copy(zbuf.at[pl.ds(0, _REM)],
                        acc.at[pl.ds(_NS * _OWN, _REM)])

    plsc.subcore_barrier()

    ebase = wid * _EPW

    def body(c, carry):
        base = pl.multiple_of(ebase + c * _CHUNK, 8)
        pltpu.sync_copy(src_hbm.at[pl.ds(base, _CHUNK)], sidx)
        pltpu.sync_copy(dst_hbm.at[pl.ds(base, _CHUNK)], didx)
        pltpu.async_copy(h_hbm.at[sidx], rows, sem).wait()
        pltpu.sync_copy(rows, acc.at[didx], add=True)
        return carry

    lax.fori_loop(0, _NCHUNK, body, 0)
    plsc.subcore_barrier()

    obase = cid * _N + rbase
    for j in range(_OWN // _WB):
        aoff = pl.multiple_of(rbase + j * _WB, 8)
        ooff = pl.multiple_of(obase + j * _WB, 8)
        pltpu.sync_copy(acc.at[pl.ds(aoff, _WB)], zbuf)
        pltpu.sync_copy(zbuf, out_hbm.at[pl.ds(ooff, _WB)])

    @pl.when(sid == _NS - 1)
    def _wb_rem():
        pltpu.sync_copy(acc.at[pl.ds(_NS * _OWN, _REM)], zbuf.at[pl.ds(0, _REM)])
        ooff = pl.multiple_of(cid * _N + _NS * _OWN, 8)
        pltpu.sync_copy(zbuf.at[pl.ds(0, _REM)], out_hbm.at[pl.ds(ooff, _REM)])


_segsum = functools.partial(
    pl.kernel,
    out_type=jax.ShapeDtypeStruct((_NC * _N, _D), jnp.float32),
    mesh=plsc.VectorSubcoreMesh(core_axis_name="c", subcore_axis_name="s"),
    scratch_types=[
        pltpu.VMEM((_CHUNK,), jnp.int32),
        pltpu.VMEM((_CHUNK,), jnp.int32),
        pltpu.VMEM((_CHUNK, _D), jnp.float32),
        pltpu.VMEM((_WB, _D), jnp.float32),
        pltpu.VMEM_SHARED((_N, _D), jnp.float32),
        pltpu.SemaphoreType.DMA,
    ],
)(_seg_body)


def _mlp1_body(s_ref, x_ref, p_ref, W1_ref, b1_ref, g1_ref, be1_ref, W2_ref,
               b2_ref, o_ref):
    h = x_ref[...] * s_ref[0, 0] + p_ref[0] + p_ref[1]
    z = jnp.dot(h, W1_ref[...], preferred_element_type=jnp.float32) + b1_ref[...]
    mu = jnp.mean(z, axis=0, keepdims=True)
    zc = z - mu
    var = jnp.mean(zc * zc, axis=0, keepdims=True)
    z = zc * lax.rsqrt(var + 1e-5) * g1_ref[...] + be1_ref[...]
    z = jnp.maximum(z, 0.0)
    z = jnp.dot(z, W2_ref[...], preferred_element_type=jnp.float32) + b2_ref[...]
    o_ref[...] = jnp.maximum(z, 0.0)


def _mlp2_body(s_ref, h_ref, p_ref, W3_ref, b3_ref, g2_ref, be2_ref, W4_ref,
               b4_ref, g3_ref, be3_ref, Wfc_ref, bfc_ref, o_ref):
    h = h_ref[...] * s_ref[0, 0] + p_ref[0] + p_ref[1]
    z = jnp.dot(h, W3_ref[...], preferred_element_type=jnp.float32) + b3_ref[...]
    mu = jnp.mean(z, axis=0, keepdims=True)
    zc = z - mu
    var = jnp.mean(zc * zc, axis=0, keepdims=True)
    z = zc * lax.rsqrt(var + 1e-5) * g2_ref[...] + be2_ref[...]
    z = jnp.maximum(z, 0.0)
    z = jnp.dot(z, W4_ref[...], preferred_element_type=jnp.float32) + b4_ref[...]
    mu2 = jnp.mean(z, axis=0, keepdims=True)
    zc2 = z - mu2
    var2 = jnp.mean(zc2 * zc2, axis=0, keepdims=True)
    z = zc2 * lax.rsqrt(var2 + 1e-5) * g3_ref[...] + be3_ref[...]
    z = jnp.maximum(z, 0.0)
    o_ref[...] = (jnp.dot(z, Wfc_ref[...], preferred_element_type=jnp.float32)
                  + bfc_ref[...])


def kernel(x, edge_index, eps1, W1, b1, g1, be1, W2, b2, eps2, W3, b3, g2,
           be2, W4, b4, g3, be3, Wfc, bfc):
    src = edge_index[0]
    dst = edge_index[1]

    p1 = _segsum(x, src, dst).reshape(2, _N, _D)
    h1 = pl.pallas_call(
        _mlp1_body,
        out_shape=jax.ShapeDtypeStruct((_N, _D), jnp.float32),
    )(
        (1.0 + eps1).reshape(1, 1), x, p1, W1, b1.reshape(1, _D),
        g1.reshape(1, _D), be1.reshape(1, _D), W2, b2.reshape(1, _D),
    )

    p2 = _segsum(h1, src, dst).reshape(2, _N, _D)
    out = pl.pallas_call(
        _mlp2_body,
        out_shape=jax.ShapeDtypeStruct((_N, Wfc.shape[1]), jnp.float32),
    )(
        (1.0 + eps2).reshape(1, 1), h1, p2, W3, b3.reshape(1, _D),
        g2.reshape(1, _D), be2.reshape(1, _D), W4, b4.reshape(1, _D),
        g3.reshape(1, _D), be3.reshape(1, _D), Wfc,
        bfc.reshape(1, bfc.shape[0]),
    )
    return out
